# baseline (speedup 1.0000x reference)
import functools

import jax
import jax.numpy as jnp
from jax import lax
from jax.experimental import pallas as pl
from jax.experimental.pallas import tpu as pltpu
from jax.experimental.pallas import tpu_sc as plsc


_ROWS = 8192
_COLS = 1024
_NC = 2
_NS = 16
_NW = _NC * _NS
_ROWS_PER_W = _ROWS // _NW


def _sc_copy(x_hbm, out_hbm):
    wid = lax.axis_index("s") * _NC + lax.axis_index("c")
    base = wid * _ROWS_PER_W
    pltpu.sync_copy(
        x_hbm.at[pl.ds(base, _ROWS_PER_W), :],
        out_hbm.at[pl.ds(base, _ROWS_PER_W), :],
    )


def kernel(x):
    mesh = plsc.VectorSubcoreMesh(core_axis_name="c", subcore_axis_name="s")
    gathered = pl.kernel(
        _sc_copy,
        out_type=jax.ShapeDtypeStruct((_ROWS, _COLS), jnp.float32),
        mesh=mesh,
    )(x)
    sizes = jnp.array([_ROWS], dtype=jnp.int32)
    return (gathered, sizes)


# speedup vs baseline: 23.0326x; 23.0326x over previous
import functools

import jax
import jax.numpy as jnp
from jax import lax
from jax.experimental import pallas as pl
from jax.experimental.pallas import tpu as pltpu
from jax.experimental.pallas import tpu_sc as plsc


_ROWS = 8192
_COLS = 1024
_NC = 2
_NS = 16
_NW = _NC * _NS
_ROWS_PER_W = _ROWS // _NW
_CHUNK = 32
_N_CHUNKS = _ROWS_PER_W // _CHUNK


def _sc_copy(x_hbm, out_hbm, buf0, buf1, lsem, ssem):
    wid = lax.axis_index("s") * _NC + lax.axis_index("c")
    base = wid * _ROWS_PER_W
    bufs = (buf0, buf1)

    def load(i):
        return pltpu.make_async_copy(
            x_hbm.at[pl.ds(base + i * _CHUNK, _CHUNK), :],
            bufs[i % 2],
            lsem.at[i % 2],
        )

    def store(i):
        return pltpu.make_async_copy(
            bufs[i % 2],
            out_hbm.at[pl.ds(base + i * _CHUNK, _CHUNK), :],
            ssem.at[i % 2],
        )

    load(0).start()
    for i in range(_N_CHUNKS):
        if i + 1 < _N_CHUNKS:
            if i - 1 >= 0:
                store(i - 1).wait()
            load(i + 1).start()
        load(i).wait()
        store(i).start()
    store(_N_CHUNKS - 2).wait()
    store(_N_CHUNKS - 1).wait()


def kernel(x):
    mesh = plsc.VectorSubcoreMesh(core_axis_name="c", subcore_axis_name="s")
    gathered = pl.kernel(
        _sc_copy,
        out_type=jax.ShapeDtypeStruct((_ROWS, _COLS), jnp.float32),
        mesh=mesh,
        scratch_types=[
            pltpu.VMEM((_CHUNK, _COLS), jnp.float32),
            pltpu.VMEM((_CHUNK, _COLS), jnp.float32),
            pltpu.SemaphoreType.DMA((2,)),
            pltpu.SemaphoreType.DMA((2,)),
        ],
    )(x)
    sizes = jnp.array([_ROWS], dtype=jnp.int32)
    return (gathered, sizes)


# speedup vs baseline: 39.9041x; 1.7325x over previous
import jax
import jax.numpy as jnp
from jax.experimental import pallas as pl
from jax.experimental.pallas import tpu as pltpu


_ROWS = 8192
_COLS = 1024
_NB = 8
_CHUNK = 128
_N_CHUNKS = _ROWS // _CHUNK
_LAG = 4


def _copy_kernel(x_hbm, o_hbm, bufs, lsem, ssem):
    def load(i):
        return pltpu.make_async_copy(
            x_hbm.at[pl.ds(i * _CHUNK, _CHUNK), :],
            bufs.at[i % _NB],
            lsem.at[i % _NB],
        )

    def store(i):
        return pltpu.make_async_copy(
            bufs.at[i % _NB],
            o_hbm.at[pl.ds(i * _CHUNK, _CHUNK), :],
            ssem.at[i % _NB],
        )

    for b in range(_NB):
        load(b).start()
    for i in range(_N_CHUNKS):
        j = i - _LAG
        if j >= 0:
            store(j).wait()
            if j + _NB < _N_CHUNKS:
                load(j + _NB).start()
        load(i).wait()
        store(i).start()
    for i in range(_N_CHUNKS - _LAG, _N_CHUNKS):
        store(i).wait()


def kernel(x):
    gathered = pl.pallas_call(
        _copy_kernel,
        in_specs=[pl.BlockSpec(memory_space=pl.ANY)],
        out_specs=pl.BlockSpec(memory_space=pl.ANY),
        out_shape=jax.ShapeDtypeStruct((_ROWS, _COLS), x.dtype),
        scratch_shapes=[
            pltpu.VMEM((_NB, _CHUNK, _COLS), jnp.float32),
            pltpu.SemaphoreType.DMA((_NB,)),
            pltpu.SemaphoreType.DMA((_NB,)),
        ],
    )(x)
    sizes = jnp.array([_ROWS], dtype=jnp.int32)
    return (gathered, sizes)
